# baseline (speedup 1.0000x reference)
import functools

import jax
import jax.numpy as jnp
from jax import lax
from jax.experimental import pallas as pl
from jax.experimental.pallas import tpu as pltpu
from jax.experimental.pallas import tpu_sc as plsc


_SPLITS = (40, 88)
_NCHUNK = len(_SPLITS)


def _build(B, V, D):
    info = plsc.get_sparse_core_info()
    NC, NS = info.num_cores, info.num_subcores
    NW = NC * NS
    b_per_w = B // NW
    assert B % NW == 0 and sum(_SPLITS) == b_per_w
    offs = [sum(_SPLITS[:c]) for c in range(_NCHUNK)]
    mesh = plsc.VectorSubcoreMesh(core_axis_name="c", subcore_axis_name="s")

    @functools.partial(
        pl.kernel,
        mesh=mesh,
        out_type=jax.ShapeDtypeStruct((B, D), jnp.float32),
        scratch_types=[
            pltpu.VMEM((b_per_w,), jnp.int32),
            pltpu.VMEM((b_per_w, D), jnp.float32),
        ]
        + [pltpu.SemaphoreType.DMA] * (2 * _NCHUNK)
        + [pltpu.SemaphoreType.DMA],
    )
    def emb(idx_hbm, table_hbm, out_hbm, idx_v, rows_v, *sems):
        isems, gsems, wsem = sems[:_NCHUNK], sems[_NCHUNK:2 * _NCHUNK], sems[-1]
        wid = lax.axis_index("s") * NC + lax.axis_index("c")
        base = wid * b_per_w
        idx_loads = [
            pltpu.async_copy(
                idx_hbm.at[pl.ds(base + offs[c], _SPLITS[c])],
                idx_v.at[pl.ds(offs[c], _SPLITS[c])],
                isems[c],
            )
            for c in range(_NCHUNK)
        ]
        gathers = []
        for c in range(_NCHUNK):
            idx_loads[c].wait()
            gathers.append(
                pltpu.async_copy(
                    table_hbm.at[idx_v.at[pl.ds(offs[c], _SPLITS[c])]],
                    rows_v.at[pl.ds(offs[c], _SPLITS[c])],
                    gsems[c],
                )
            )
        writes = []
        for c in range(_NCHUNK):
            gathers[c].wait()
            writes.append(
                pltpu.async_copy(
                    rows_v.at[pl.ds(offs[c], _SPLITS[c])],
                    out_hbm.at[pl.ds(base + offs[c], _SPLITS[c])],
                    wsem,
                )
            )
        for w in writes:
            w.wait()

    return emb


def kernel(labels, embedding_table):
    B, = labels.shape
    V, D = embedding_table.shape
    emb = _build(B, V, D)
    return emb(labels.astype(jnp.int32), embedding_table)


# speedup vs baseline: 1.0023x; 1.0023x over previous
import functools

import jax
import jax.numpy as jnp
from jax import lax
from jax.experimental import pallas as pl
from jax.experimental.pallas import tpu as pltpu
from jax.experimental.pallas import tpu_sc as plsc


_SPLITS = (64, 64)
_NCHUNK = len(_SPLITS)


def _build(B, V, D):
    info = plsc.get_sparse_core_info()
    NC, NS = info.num_cores, info.num_subcores
    NW = NC * NS
    b_per_w = B // NW
    assert B % NW == 0 and sum(_SPLITS) == b_per_w
    offs = [sum(_SPLITS[:c]) for c in range(_NCHUNK)]
    mesh = plsc.VectorSubcoreMesh(core_axis_name="c", subcore_axis_name="s")

    @functools.partial(
        pl.kernel,
        mesh=mesh,
        out_type=jax.ShapeDtypeStruct((B, D), jnp.float32),
        scratch_types=[
            pltpu.VMEM((b_per_w,), jnp.int32),
            pltpu.VMEM((b_per_w, D), jnp.float32),
        ]
        + [pltpu.SemaphoreType.DMA] * (2 * _NCHUNK)
        + [pltpu.SemaphoreType.DMA],
    )
    def emb(idx_hbm, table_hbm, out_hbm, idx_v, rows_v, *sems):
        isems, gsems, wsem = sems[:_NCHUNK], sems[_NCHUNK:2 * _NCHUNK], sems[-1]
        wid = lax.axis_index("s") * NC + lax.axis_index("c")
        base = wid * b_per_w
        idx_loads = [
            pltpu.async_copy(
                idx_hbm.at[pl.ds(base + offs[c], _SPLITS[c])],
                idx_v.at[pl.ds(offs[c], _SPLITS[c])],
                isems[c],
            )
            for c in range(_NCHUNK)
        ]
        gathers = []
        for c in range(_NCHUNK):
            idx_loads[c].wait()
            gathers.append(
                pltpu.async_copy(
                    table_hbm.at[idx_v.at[pl.ds(offs[c], _SPLITS[c])]],
                    rows_v.at[pl.ds(offs[c], _SPLITS[c])],
                    gsems[c],
                )
            )
        writes = []
        for c in range(_NCHUNK):
            gathers[c].wait()
            writes.append(
                pltpu.async_copy(
                    rows_v.at[pl.ds(offs[c], _SPLITS[c])],
                    out_hbm.at[pl.ds(base + offs[c], _SPLITS[c])],
                    wsem,
                )
            )
        for w in writes:
            w.wait()

    return emb


def kernel(labels, embedding_table):
    B, = labels.shape
    V, D = embedding_table.shape
    emb = _build(B, V, D)
    return emb(labels.astype(jnp.int32), embedding_table)
